# baseline (speedup 1.0000x reference)
import functools

import jax
import jax.numpy as jnp
from jax import lax
from jax.experimental import pallas as pl
from jax.experimental.pallas import tpu as pltpu
from jax.experimental.pallas import tpu_sc as plsc

SCALE = 511.0
B, C, H, W = 2, 128, 512, 512
HW = H * W
N = 131072

NC, NS, L = 2, 16, 16
NW = NC * NS
PTS_PER_W = (B * N) // NW
P = 128
CHUNKS = PTS_PER_W // P

HCHUNK = 4096
NCHUNK = 2048


def _tr_in_body(g_ref, t_ref):
    t_ref[...] = g_ref[0].T


def _build_table(grid3):
    nblk = HW // HCHUNK
    return pl.pallas_call(
        _tr_in_body,
        grid=(B, nblk),
        in_specs=[pl.BlockSpec((1, C, HCHUNK), lambda b, j: (b, 0, j))],
        out_specs=pl.BlockSpec((HCHUNK, C), lambda b, j: (b * nblk + j, 0)),
        out_shape=jax.ShapeDtypeStruct((B * HW, C), jnp.float32),
    )(grid3)


def _tr_out_body(p_ref, o_ref):
    o_ref[0] = p_ref[0].T


def _transpose_out(pm):
    nblk = N // NCHUNK
    return pl.pallas_call(
        _tr_out_body,
        grid=(B, nblk),
        in_specs=[pl.BlockSpec((1, NCHUNK, C), lambda b, j: (b, j, 0))],
        out_specs=pl.BlockSpec((1, C, NCHUNK), lambda b, j: (b, 0, j)),
        out_shape=jax.ShapeDtypeStruct((B, C, N), jnp.float32),
    )(pm)


@functools.partial(
    pl.kernel,
    out_type=jax.ShapeDtypeStruct((B, N, C), jnp.float32),
    mesh=plsc.VectorSubcoreMesh(core_axis_name="c", subcore_axis_name="s"),
    compiler_params=pltpu.CompilerParams(needs_layout_passes=False),
    scratch_types=[
        pltpu.VMEM((P,), jnp.float32),
        pltpu.VMEM((P,), jnp.float32),
        pltpu.VMEM((P,), jnp.int32),
        pltpu.VMEM((P,), jnp.int32),
        pltpu.VMEM((P,), jnp.int32),
        pltpu.VMEM((P,), jnp.int32),
        pltpu.VMEM((P,), jnp.float32),
        pltpu.VMEM((P,), jnp.float32),
        pltpu.VMEM((P,), jnp.float32),
        pltpu.VMEM((P,), jnp.float32),
        pltpu.VMEM((P, C), jnp.float32),
        pltpu.VMEM((P, C), jnp.float32),
        pltpu.VMEM((P, C), jnp.float32),
        pltpu.VMEM((P, C), jnp.float32),
        pltpu.VMEM((P, C), jnp.float32),
        pltpu.SemaphoreType.DMA,
    ],
)
def _sc_gather(table, h_hbm, w_hbm, out, h_v, w_v, i00, i01, i10, i11,
               w00, w01, w10, w11, r00, r01, r10, r11, opm, sem):
    cid = lax.axis_index("c")
    sid = lax.axis_index("s")
    wid = sid * NC + cid
    b = wid // NS
    lane = wid % NS
    base = lane * PTS_PER_W
    boff = b * HW

    def chunk(g, carry):
        n0 = base + g * P
        pltpu.sync_copy(h_hbm.at[b, pl.ds(n0, P)], h_v)
        pltpu.sync_copy(w_hbm.at[b, pl.ds(n0, P)], w_v)
        for t in range(P // L):
            sl = pl.ds(t * L, L)
            hv = h_v[sl] * SCALE
            wv = w_v[sl] * SCALE
            h0i = hv.astype(jnp.int32)
            w0i = wv.astype(jnp.int32)
            wh1 = hv - h0i.astype(jnp.float32)
            ww1 = wv - w0i.astype(jnp.float32)
            wh0 = 1.0 - wh1
            ww0 = 1.0 - ww1
            r0 = boff + h0i * W + w0i
            i00[sl] = r0
            i01[sl] = r0 + 1
            i10[sl] = r0 + W
            i11[sl] = r0 + (W + 1)
            w00[sl] = wh0 * ww0
            w01[sl] = wh0 * ww1
            w10[sl] = wh1 * ww0
            w11[sl] = wh1 * ww1
        d1 = pltpu.async_copy(table.at[i00], r00, sem)
        d2 = pltpu.async_copy(table.at[i01], r01, sem)
        d3 = pltpu.async_copy(table.at[i10], r10, sem)
        d4 = pltpu.async_copy(table.at[i11], r11, sem)
        d1.wait()
        d2.wait()
        d3.wait()
        d4.wait()

        def pt(i, carry2):
            iv = jnp.full((L,), i, jnp.int32)
            a00 = plsc.load_gather(w00, [iv])
            a01 = plsc.load_gather(w01, [iv])
            a10 = plsc.load_gather(w10, [iv])
            a11 = plsc.load_gather(w11, [iv])
            for t in range(C // L):
                sl = pl.ds(t * L, L)
                opm[i, sl] = (r00[i, sl] * a00 + r01[i, sl] * a01
                              + r10[i, sl] * a10 + r11[i, sl] * a11)
            return carry2

        lax.fori_loop(0, P, pt, 0)
        pltpu.sync_copy(opm, out.at[b, pl.ds(n0, P), :])
        return carry

    lax.fori_loop(0, CHUNKS, chunk, 0)


def kernel(grid_in, pcds_ind):
    grid3 = grid_in.reshape(B, C, HW)
    table = _build_table(grid3)
    coords = pcds_ind[..., 0]
    h = coords[..., 0]
    w = coords[..., 1]
    pm = _sc_gather(table, h, w)
    out = _transpose_out(pm)
    return out[..., None]
